# baseline (speedup 1.0000x reference)
import functools

import jax
import jax.numpy as jnp
from jax import lax
from jax.experimental import pallas as pl
from jax.experimental.pallas import tpu as pltpu
from jax.experimental.pallas import tpu_sc as plsc

B, S, D = 2, 2048, 768
E, K, H = 64, 2, 512
T = B * S
NA = T * K
BS = 128
NB = NA // BS + E
NPAD = NB * BS

_F32 = jnp.float32
_I32 = jnp.int32


_TB = 512
_NTB = T // _TB


def _softplus(z):
    return jnp.maximum(z, 0.0) + jnp.log1p(jnp.exp(-jnp.abs(z)))


def _gating_body(x_ref, nz_ref, wg_ref, bg_ref, wn_ref, bn_ref,
                 i0_ref, i1_ref, r0_ref, r1_ref, w0_ref, w1_ref,
                 cnt_ref, gp_ref):
    b = pl.program_id(0)
    x = x_ref[...]
    logits = jnp.dot(x, wg_ref[...], preferred_element_type=_F32) + bg_ref[...]
    sp = _softplus(jnp.dot(x, wn_ref[...], preferred_element_type=_F32)
                   + bn_ref[...])
    noisy = logits + nz_ref[...] * sp

    ie = lax.broadcasted_iota(_I32, (_TB, E), 1)
    m1 = jnp.max(noisy, axis=1, keepdims=True)
    i0 = jnp.min(jnp.where(noisy == m1, ie, E), axis=1, keepdims=True)
    noisy2 = jnp.where(ie == i0, -jnp.inf, noisy)
    m2 = jnp.max(noisy2, axis=1, keepdims=True)
    i1 = jnp.min(jnp.where(noisy2 == m2, ie, E), axis=1, keepdims=True)

    d = jnp.exp(m2 - m1)
    w0 = 1.0 / (1.0 + d)
    w1 = 1.0 - w0

    a0 = (ie == i0).astype(_F32)
    a1 = (ie == i1).astype(_F32)
    s = a0 + a1

    gmax = jnp.max(logits, axis=1, keepdims=True)
    ge = jnp.exp(logits - gmax)
    gp = ge / jnp.sum(ge, axis=1, keepdims=True)

    @pl.when(b == 0)
    def _():
        cnt_ref[...] = jnp.zeros_like(cnt_ref)
        gp_ref[...] = jnp.zeros_like(gp_ref)

    carry = cnt_ref[...]
    it = lax.broadcasted_iota(_I32, (_TB, _TB), 0)
    iu = lax.broadcasted_iota(_I32, (_TB, _TB), 1)
    lstrict = (iu < it).astype(_F32)
    p = jnp.dot(lstrict, s, preferred_element_type=_F32,
                precision=lax.Precision.HIGHEST) + carry
    r0 = jnp.sum(p * a0, axis=1, keepdims=True)
    r1 = jnp.sum(p * a1, axis=1, keepdims=True)

    cnt_ref[...] = carry + jnp.sum(s, axis=0, keepdims=True)
    gp_ref[...] += jnp.sum(gp, axis=0, keepdims=True)
    i0_ref[...] = i0
    i1_ref[...] = i1
    r0_ref[...] = r0.astype(_I32)
    r1_ref[...] = r1.astype(_I32)
    w0_ref[...] = w0
    w1_ref[...] = w1


def _gating(xf, nz, Wg, bg, Wn, bn, interpret=False):
    v1 = pl.BlockSpec((T // _NTB, 1), lambda i: (i, 0))
    out = pl.pallas_call(
        _gating_body,
        grid=(_NTB,),
        in_specs=[
            pl.BlockSpec((_TB, D), lambda i: (i, 0)),
            pl.BlockSpec((_TB, E), lambda i: (i, 0)),
            pl.BlockSpec((D, E), lambda i: (0, 0)),
            pl.BlockSpec((1, E), lambda i: (0, 0)),
            pl.BlockSpec((D, E), lambda i: (0, 0)),
            pl.BlockSpec((1, E), lambda i: (0, 0)),
        ],
        out_specs=[v1, v1, v1, v1, v1, v1,
                   pl.BlockSpec((1, E), lambda i: (0, 0)),
                   pl.BlockSpec((1, E), lambda i: (0, 0))],
        out_shape=[
            jax.ShapeDtypeStruct((T, 1), _I32),
            jax.ShapeDtypeStruct((T, 1), _I32),
            jax.ShapeDtypeStruct((T, 1), _I32),
            jax.ShapeDtypeStruct((T, 1), _I32),
            jax.ShapeDtypeStruct((T, 1), _F32),
            jax.ShapeDtypeStruct((T, 1), _F32),
            jax.ShapeDtypeStruct((1, E), _F32),
            jax.ShapeDtypeStruct((1, E), _F32),
        ],
        interpret=interpret,
    )(xf, nz, Wg, bg.reshape(1, E), Wn, bn.reshape(1, E))
    return out


def _route_body(cnt_ref, gp_ref, i0_ref, i1_ref, r0_ref, r1_ref,
                s0_ref, s1_ref, be_ref, loss_ref):
    cnt = cnt_ref[...]
    nb = jnp.floor((cnt + (BS - 1)) * (1.0 / BS))
    ie1 = lax.broadcasted_iota(_I32, (E, E), 0)
    ie2 = lax.broadcasted_iota(_I32, (E, E), 1)
    lstrict = (ie1 < ie2).astype(_F32)
    blkbase = jnp.dot(nb, lstrict, preferred_element_type=_F32,
                      precision=lax.Precision.HIGHEST)
    padbase = blkbase * float(BS)

    ie = lax.broadcasted_iota(_I32, (T, E), 1)
    a0 = (ie == i0_ref[...]).astype(_F32)
    a1 = (ie == i1_ref[...]).astype(_F32)
    pb0 = jnp.sum(a0 * padbase, axis=1, keepdims=True)
    pb1 = jnp.sum(a1 * padbase, axis=1, keepdims=True)
    s0_ref[...] = r0_ref[...] + pb0.astype(_I32)
    s1_ref[...] = r1_ref[...] + pb1.astype(_I32)

    used = jnp.sum(nb)
    ib = lax.broadcasted_iota(_I32, (1, NB), 1).astype(_F32)
    ibc = jnp.minimum(ib, used - 1.0)
    owns = (jnp.broadcast_to(blkbase.reshape(E, 1), (E, NB))
            <= jnp.broadcast_to(ibc, (E, NB)))
    be = jnp.sum(owns.astype(_F32), axis=0, keepdims=True) - 1.0
    flag = (ib < used)
    be_ref[...] = jnp.concatenate(
        [be.astype(_I32), flag.astype(_I32), ibc.astype(_I32)], axis=0)

    gp = gp_ref[...]
    loss_ref[...] = jnp.sum(cnt * gp).reshape(1, 1) * (
        float(E) / (float(NA) * float(T)))


def _route(cnt, gp, i0, i1, r0, r1, interpret=False):
    return pl.pallas_call(
        _route_body,
        out_shape=[
            jax.ShapeDtypeStruct((T, 1), _I32),
            jax.ShapeDtypeStruct((T, 1), _I32),
            jax.ShapeDtypeStruct((3, NB), _I32),
            jax.ShapeDtypeStruct((1, 1), _F32),
        ],
        interpret=interpret,
    )(cnt, gp, i0, i1, r0, r1)


def _ffn_body(be_ref, xs_ref, ws_ref, w1_ref, b1_ref, w2_ref, b2_ref,
              ys_ref):
    @pl.when(be_ref[1, pl.program_id(0)] == 1)
    def _():
        xb = xs_ref[...]
        h = jnp.dot(xb, w1_ref[0], preferred_element_type=_F32) + b1_ref[0]
        h = 0.5 * h * (1.0 + lax.erf(h * 0.7071067811865476))
        y = jnp.dot(h, w2_ref[0], preferred_element_type=_F32) + b2_ref[0]
        ys_ref[...] = y * ws_ref[...]


def _ffn(be, xs, ws, W1, b1, W2, b2, interpret=False):
    grid_spec = pltpu.PrefetchScalarGridSpec(
        num_scalar_prefetch=1,
        grid=(NB,),
    in_specs=[
            pl.BlockSpec((BS, D), lambda b, be: (be[2, b], 0)),
            pl.BlockSpec((BS, 1), lambda b, be: (be[2, b], 0)),
            pl.BlockSpec((1, D, H), lambda b, be: (be[0, b], 0, 0)),
            pl.BlockSpec((1, 1, H), lambda b, be: (be[0, b], 0, 0)),
            pl.BlockSpec((1, H, D), lambda b, be: (be[0, b], 0, 0)),
            pl.BlockSpec((1, 1, D), lambda b, be: (be[0, b], 0, 0)),
        ],
        out_specs=pl.BlockSpec((BS, D), lambda b, be: (be[2, b], 0)),
    )
    return pl.pallas_call(
        _ffn_body,
        grid_spec=grid_spec,
        out_shape=jax.ShapeDtypeStruct((NPAD, D), _F32),
        interpret=interpret,
    )(be, xs, ws, W1, b1.reshape(E, 1, H), W2, b2.reshape(E, 1, D))


_NW = 32
_CHUNK = 64


_TPW = T // _NW


def _dispatch_body(xf_h, s0_h, s1_h, w0_h, w1_h, xs_h, ws_h,
                   i0_v, i1_v, wv0, wv1, rows_v,
                   sem_m, sem_r, sem_s):
    wid = lax.axis_index("s") * 2 + lax.axis_index("c")
    base = wid * _TPW
    sl = pl.ds(base, _TPW)
    meta = [pltpu.async_copy(s0_h.at[sl], i0_v.at[0], sem_m),
            pltpu.async_copy(s1_h.at[sl], i1_v.at[0], sem_m),
            pltpu.async_copy(w0_h.at[sl], wv0.at[0], sem_m),
            pltpu.async_copy(w1_h.at[sl], wv1.at[0], sem_m)]
    r0 = pltpu.async_copy(xf_h.at[sl], rows_v, sem_r)
    for m in meta:
        m.wait()
    r0.wait()
    scat = [pltpu.async_copy(rows_v, xs_h.at[i0_v.at[0]], sem_s),
            pltpu.async_copy(rows_v, xs_h.at[i1_v.at[0]], sem_s),
            pltpu.async_copy(wv0.at[0], ws_h.at[i0_v.at[0]], sem_s),
            pltpu.async_copy(wv1.at[0], ws_h.at[i1_v.at[0]], sem_s)]
    for s in scat:
        s.wait()


def _dispatch():
    return pl.kernel(
        _dispatch_body,
        out_type=[
            jax.ShapeDtypeStruct((NPAD, D), _F32),
            jax.ShapeDtypeStruct((NPAD,), _F32),
        ],
        mesh=plsc.VectorSubcoreMesh(core_axis_name="c",
                                    subcore_axis_name="s"),
        scratch_types=[
            pltpu.VMEM((1, _TPW), _I32),
            pltpu.VMEM((1, _TPW), _I32),
            pltpu.VMEM((1, _TPW), _F32),
            pltpu.VMEM((1, _TPW), _F32),
            pltpu.VMEM((_TPW, D), _F32),
            pltpu.SemaphoreType.DMA,
            pltpu.SemaphoreType.DMA,
            pltpu.SemaphoreType.DMA,
        ],
    )


def _combine_body(s0_h, s1_h, ys_h, out_h, i0_v, i1_v, acc_v, r1_v,
                  sem_m, sem_g, sem_o):
    wid = lax.axis_index("s") * 2 + lax.axis_index("c")
    base = wid * (T // _NW)
    nc = T // (_NW * _CHUNK)
    meta = []
    for c in range(nc):
        sl = pl.ds(base + c * _CHUNK, _CHUNK)
        meta += [pltpu.async_copy(s0_h.at[sl], i0_v.at[c], sem_m),
                 pltpu.async_copy(s1_h.at[sl], i1_v.at[c], sem_m)]
    for m in meta:
        m.wait()

    def _addrows(lo):
        def _addrow(j, carry):
            for dd in range(D // 16):
                sl = pl.ds(dd * 16, 16)
                acc_v[j, sl] = acc_v[j, sl] + r1_v[j, sl]
            return carry
        lax.fori_loop(lo, lo + _CHUNK // 2, _addrow, 0)

    for c in range(nc):
        g0 = pltpu.async_copy(ys_h.at[i0_v.at[c]], acc_v, sem_g)
        g1 = pltpu.async_copy(ys_h.at[i1_v.at[c]], r1_v, sem_g)
        g0.wait()
        g1.wait()
        _addrows(0)
        _addrows(_CHUNK // 2)
        pltpu.async_copy(acc_v, out_h.at[pl.ds(base + c * _CHUNK,
                                               _CHUNK)], sem_o).wait()


def _combine():
    return pl.kernel(
        _combine_body,
        out_type=jax.ShapeDtypeStruct((T, D), _F32),
        mesh=plsc.VectorSubcoreMesh(core_axis_name="c",
                                    subcore_axis_name="s"),
        scratch_types=[
            pltpu.VMEM((2, _CHUNK), _I32),
            pltpu.VMEM((2, _CHUNK), _I32),
            pltpu.VMEM((_CHUNK, D), _F32),
            pltpu.VMEM((_CHUNK, D), _F32),
            pltpu.SemaphoreType.DMA,
            pltpu.SemaphoreType.DMA,
            pltpu.SemaphoreType.DMA,
        ],
    )


def kernel(x, Wg, bg, Wn, bn, W1, b1, W2, b2):
    xf = x.reshape(T, D)
    nz = jax.random.normal(jax.random.key(42), (B, S, E),
                           dtype=_F32).reshape(T, E)
    i0, i1, r0, r1, w0, w1, cnt, gp = _gating(xf, nz, Wg, bg, Wn, bn)
    s0, s1, be, loss = _route(cnt, gp, i0, i1, r0, r1)
    s0f = s0.reshape(T)
    s1f = s1.reshape(T)
    xs, ws = _dispatch()(xf, s0f, s1f, w0.reshape(T), w1.reshape(T))
    ys = _ffn(be, xs, ws.reshape(NPAD, 1), W1, b1, W2, b2)
    outf = _combine()(s0f, s1f, ys)
    return outf.reshape(B, S, D), loss.reshape(())


# speedup vs baseline: 1.0895x; 1.0895x over previous
import functools

import jax
import jax.numpy as jnp
from jax import lax
from jax.experimental import pallas as pl
from jax.experimental.pallas import tpu as pltpu
from jax.experimental.pallas import tpu_sc as plsc

B, S, D = 2, 2048, 768
E, K, H = 64, 2, 512
T = B * S
NA = T * K
BS = 256
NB = NA // BS + E
NPAD = NB * BS

_F32 = jnp.float32
_I32 = jnp.int32


_TB = 512
_NTB = T // _TB


def _softplus(z):
    return jnp.maximum(z, 0.0) + jnp.log1p(jnp.exp(-jnp.abs(z)))


def _gating_body(x_ref, nz_ref, wg_ref, bg_ref, wn_ref, bn_ref,
                 i0_ref, i1_ref, r0_ref, r1_ref, w0_ref, w1_ref,
                 cnt_ref, gp_ref):
    b = pl.program_id(0)
    x = x_ref[...]
    logits = jnp.dot(x, wg_ref[...], preferred_element_type=_F32) + bg_ref[...]
    sp = _softplus(jnp.dot(x, wn_ref[...], preferred_element_type=_F32)
                   + bn_ref[...])
    noisy = logits + nz_ref[...] * sp

    ie = lax.broadcasted_iota(_I32, (_TB, E), 1)
    m1 = jnp.max(noisy, axis=1, keepdims=True)
    i0 = jnp.min(jnp.where(noisy == m1, ie, E), axis=1, keepdims=True)
    noisy2 = jnp.where(ie == i0, -jnp.inf, noisy)
    m2 = jnp.max(noisy2, axis=1, keepdims=True)
    i1 = jnp.min(jnp.where(noisy2 == m2, ie, E), axis=1, keepdims=True)

    d = jnp.exp(m2 - m1)
    w0 = 1.0 / (1.0 + d)
    w1 = 1.0 - w0

    a0 = (ie == i0).astype(_F32)
    a1 = (ie == i1).astype(_F32)
    s = a0 + a1

    gmax = jnp.max(logits, axis=1, keepdims=True)
    ge = jnp.exp(logits - gmax)
    gp = ge / jnp.sum(ge, axis=1, keepdims=True)

    @pl.when(b == 0)
    def _():
        cnt_ref[...] = jnp.zeros_like(cnt_ref)
        gp_ref[...] = jnp.zeros_like(gp_ref)

    carry = cnt_ref[...]
    it = lax.broadcasted_iota(_I32, (_TB, _TB), 0)
    iu = lax.broadcasted_iota(_I32, (_TB, _TB), 1)
    lstrict = (iu < it).astype(_F32)
    p = jnp.dot(lstrict, s, preferred_element_type=_F32,
                precision=lax.Precision.HIGHEST) + carry
    r0 = jnp.sum(p * a0, axis=1, keepdims=True)
    r1 = jnp.sum(p * a1, axis=1, keepdims=True)

    cnt_ref[...] = carry + jnp.sum(s, axis=0, keepdims=True)
    gp_ref[...] += jnp.sum(gp, axis=0, keepdims=True)
    i0_ref[...] = i0
    i1_ref[...] = i1
    r0_ref[...] = r0.astype(_I32)
    r1_ref[...] = r1.astype(_I32)
    w0_ref[...] = w0
    w1_ref[...] = w1


def _gating(xf, nz, Wg, bg, Wn, bn, interpret=False):
    v1 = pl.BlockSpec((T // _NTB, 1), lambda i: (i, 0))
    out = pl.pallas_call(
        _gating_body,
        grid=(_NTB,),
        in_specs=[
            pl.BlockSpec((_TB, D), lambda i: (i, 0)),
            pl.BlockSpec((_TB, E), lambda i: (i, 0)),
            pl.BlockSpec((D, E), lambda i: (0, 0)),
            pl.BlockSpec((1, E), lambda i: (0, 0)),
            pl.BlockSpec((D, E), lambda i: (0, 0)),
            pl.BlockSpec((1, E), lambda i: (0, 0)),
        ],
        out_specs=[v1, v1, v1, v1, v1, v1,
                   pl.BlockSpec((1, E), lambda i: (0, 0)),
                   pl.BlockSpec((1, E), lambda i: (0, 0))],
        out_shape=[
            jax.ShapeDtypeStruct((T, 1), _I32),
            jax.ShapeDtypeStruct((T, 1), _I32),
            jax.ShapeDtypeStruct((T, 1), _I32),
            jax.ShapeDtypeStruct((T, 1), _I32),
            jax.ShapeDtypeStruct((T, 1), _F32),
            jax.ShapeDtypeStruct((T, 1), _F32),
            jax.ShapeDtypeStruct((1, E), _F32),
            jax.ShapeDtypeStruct((1, E), _F32),
        ],
        interpret=interpret,
    )(xf, nz, Wg, bg.reshape(1, E), Wn, bn.reshape(1, E))
    return out


def _route_body(cnt_ref, gp_ref, i0_ref, i1_ref, r0_ref, r1_ref,
                s0_ref, s1_ref, be_ref, loss_ref):
    cnt = cnt_ref[...]
    nb = jnp.floor((cnt + (BS - 1)) * (1.0 / BS))
    ie1 = lax.broadcasted_iota(_I32, (E, E), 0)
    ie2 = lax.broadcasted_iota(_I32, (E, E), 1)
    lstrict = (ie1 < ie2).astype(_F32)
    blkbase = jnp.dot(nb, lstrict, preferred_element_type=_F32,
                      precision=lax.Precision.HIGHEST)
    padbase = blkbase * float(BS)

    ie = lax.broadcasted_iota(_I32, (T, E), 1)
    a0 = (ie == i0_ref[...]).astype(_F32)
    a1 = (ie == i1_ref[...]).astype(_F32)
    pb0 = jnp.sum(a0 * padbase, axis=1, keepdims=True)
    pb1 = jnp.sum(a1 * padbase, axis=1, keepdims=True)
    s0_ref[...] = r0_ref[...] + pb0.astype(_I32)
    s1_ref[...] = r1_ref[...] + pb1.astype(_I32)

    used = jnp.sum(nb)
    ib = lax.broadcasted_iota(_I32, (1, NB), 1).astype(_F32)
    ibc = jnp.minimum(ib, used - 1.0)
    owns = (jnp.broadcast_to(blkbase.reshape(E, 1), (E, NB))
            <= jnp.broadcast_to(ibc, (E, NB)))
    be = jnp.sum(owns.astype(_F32), axis=0, keepdims=True) - 1.0
    flag = (ib < used)
    be_ref[...] = jnp.concatenate(
        [be.astype(_I32), flag.astype(_I32), ibc.astype(_I32)], axis=0)

    gp = gp_ref[...]
    loss_ref[...] = jnp.sum(cnt * gp).reshape(1, 1) * (
        float(E) / (float(NA) * float(T)))


def _route(cnt, gp, i0, i1, r0, r1, interpret=False):
    return pl.pallas_call(
        _route_body,
        out_shape=[
            jax.ShapeDtypeStruct((T, 1), _I32),
            jax.ShapeDtypeStruct((T, 1), _I32),
            jax.ShapeDtypeStruct((3, NB), _I32),
            jax.ShapeDtypeStruct((1, 1), _F32),
        ],
        interpret=interpret,
    )(cnt, gp, i0, i1, r0, r1)


def _ffn_body(be_ref, xs_ref, ws_ref, w1_ref, b1_ref, w2_ref, b2_ref,
              ys_ref):
    @pl.when(be_ref[1, pl.program_id(0)] == 1)
    def _():
        xb = xs_ref[...]
        h = jnp.dot(xb, w1_ref[0], preferred_element_type=_F32) + b1_ref[0]
        h = 0.5 * h * (1.0 + lax.erf(h * 0.7071067811865476))
        y = jnp.dot(h, w2_ref[0], preferred_element_type=_F32) + b2_ref[0]
        ys_ref[...] = y * ws_ref[...]


def _ffn(be, xs, ws, W1, b1, W2, b2, interpret=False):
    grid_spec = pltpu.PrefetchScalarGridSpec(
        num_scalar_prefetch=1,
        grid=(NB,),
    in_specs=[
            pl.BlockSpec((BS, D), lambda b, be: (be[2, b], 0)),
            pl.BlockSpec((BS, 1), lambda b, be: (be[2, b], 0)),
            pl.BlockSpec((1, D, H), lambda b, be: (be[0, b], 0, 0)),
            pl.BlockSpec((1, 1, H), lambda b, be: (be[0, b], 0, 0)),
            pl.BlockSpec((1, H, D), lambda b, be: (be[0, b], 0, 0)),
            pl.BlockSpec((1, 1, D), lambda b, be: (be[0, b], 0, 0)),
        ],
        out_specs=pl.BlockSpec((BS, D), lambda b, be: (be[2, b], 0)),
    )
    return pl.pallas_call(
        _ffn_body,
        grid_spec=grid_spec,
        out_shape=jax.ShapeDtypeStruct((NPAD, D), _F32),
        interpret=interpret,
    )(be, xs, ws, W1, b1.reshape(E, 1, H), W2, b2.reshape(E, 1, D))


_NW = 32
_CHUNK = 64


_TPW = T // _NW


def _dispatch_body(xf_h, s0_h, s1_h, w0_h, w1_h, xs_h, ws_h,
                   i0_v, i1_v, wv0, wv1, rows_v,
                   sem_m, sem_r, sem_s):
    wid = lax.axis_index("s") * 2 + lax.axis_index("c")
    base = wid * _TPW
    sl = pl.ds(base, _TPW)
    meta = [pltpu.async_copy(s0_h.at[sl], i0_v.at[0], sem_m),
            pltpu.async_copy(s1_h.at[sl], i1_v.at[0], sem_m),
            pltpu.async_copy(w0_h.at[sl], wv0.at[0], sem_m),
            pltpu.async_copy(w1_h.at[sl], wv1.at[0], sem_m)]
    r0 = pltpu.async_copy(xf_h.at[sl], rows_v, sem_r)
    for m in meta:
        m.wait()
    r0.wait()
    scat = [pltpu.async_copy(rows_v, xs_h.at[i0_v.at[0]], sem_s),
            pltpu.async_copy(rows_v, xs_h.at[i1_v.at[0]], sem_s),
            pltpu.async_copy(wv0.at[0], ws_h.at[i0_v.at[0]], sem_s),
            pltpu.async_copy(wv1.at[0], ws_h.at[i1_v.at[0]], sem_s)]
    for s in scat:
        s.wait()


def _dispatch():
    return pl.kernel(
        _dispatch_body,
        out_type=[
            jax.ShapeDtypeStruct((NPAD, D), _F32),
            jax.ShapeDtypeStruct((NPAD,), _F32),
        ],
        mesh=plsc.VectorSubcoreMesh(core_axis_name="c",
                                    subcore_axis_name="s"),
        scratch_types=[
            pltpu.VMEM((1, _TPW), _I32),
            pltpu.VMEM((1, _TPW), _I32),
            pltpu.VMEM((1, _TPW), _F32),
            pltpu.VMEM((1, _TPW), _F32),
            pltpu.VMEM((_TPW, D), _F32),
            pltpu.SemaphoreType.DMA,
            pltpu.SemaphoreType.DMA,
            pltpu.SemaphoreType.DMA,
        ],
    )


def _combine_body(s0_h, s1_h, ys_h, out_h, i0_v, i1_v, acc_v, r1_v,
                  sem_m, sem_g, sem_o):
    wid = lax.axis_index("s") * 2 + lax.axis_index("c")
    base = wid * (T // _NW)
    nc = T // (_NW * _CHUNK)
    meta = []
    for c in range(nc):
        sl = pl.ds(base + c * _CHUNK, _CHUNK)
        meta += [pltpu.async_copy(s0_h.at[sl], i0_v.at[c], sem_m),
                 pltpu.async_copy(s1_h.at[sl], i1_v.at[c], sem_m)]
    for m in meta:
        m.wait()

    def _addrows(lo):
        def _addrow(j, carry):
            for dd in range(D // 16):
                sl = pl.ds(dd * 16, 16)
                acc_v[j, sl] = acc_v[j, sl] + r1_v[j, sl]
            return carry
        lax.fori_loop(lo, lo + _CHUNK // 2, _addrow, 0)

    for c in range(nc):
        g0 = pltpu.async_copy(ys_h.at[i0_v.at[c]], acc_v, sem_g)
        g1 = pltpu.async_copy(ys_h.at[i1_v.at[c]], r1_v, sem_g)
        g0.wait()
        g1.wait()
        _addrows(0)
        _addrows(_CHUNK // 2)
        pltpu.async_copy(acc_v, out_h.at[pl.ds(base + c * _CHUNK,
                                               _CHUNK)], sem_o).wait()


def _combine():
    return pl.kernel(
        _combine_body,
        out_type=jax.ShapeDtypeStruct((T, D), _F32),
        mesh=plsc.VectorSubcoreMesh(core_axis_name="c",
                                    subcore_axis_name="s"),
        scratch_types=[
            pltpu.VMEM((2, _CHUNK), _I32),
            pltpu.VMEM((2, _CHUNK), _I32),
            pltpu.VMEM((_CHUNK, D), _F32),
            pltpu.VMEM((_CHUNK, D), _F32),
            pltpu.SemaphoreType.DMA,
            pltpu.SemaphoreType.DMA,
            pltpu.SemaphoreType.DMA,
        ],
    )


def kernel(x, Wg, bg, Wn, bn, W1, b1, W2, b2):
    xf = x.reshape(T, D)
    nz = jax.random.normal(jax.random.key(42), (B, S, E),
                           dtype=_F32).reshape(T, E)
    i0, i1, r0, r1, w0, w1, cnt, gp = _gating(xf, nz, Wg, bg, Wn, bn)
    s0, s1, be, loss = _route(cnt, gp, i0, i1, r0, r1)
    s0f = s0.reshape(T)
    s1f = s1.reshape(T)
    xs, ws = _dispatch()(xf, s0f, s1f, w0.reshape(T), w1.reshape(T))
    ys = _ffn(be, xs, ws.reshape(NPAD, 1), W1, b1, W2, b2)
    outf = _combine()(s0f, s1f, ys)
    return outf.reshape(B, S, D), loss.reshape(())


# speedup vs baseline: 1.0926x; 1.0028x over previous
import functools

import jax
import jax.numpy as jnp
from jax import lax
from jax.experimental import pallas as pl
from jax.experimental.pallas import tpu as pltpu
from jax.experimental.pallas import tpu_sc as plsc

B, S, D = 2, 2048, 768
E, K, H = 64, 2, 512
T = B * S
NA = T * K
BS = 256
NB = NA // BS + E
NPAD = NB * BS

_F32 = jnp.float32
_I32 = jnp.int32


_TB = 512
_NTB = T // _TB


def _softplus(z):
    return jnp.maximum(z, 0.0) + jnp.log1p(jnp.exp(-jnp.abs(z)))


def _gating_body(x_ref, nz_ref, wgn_ref, bgn_ref,
                 w0_ref, w1_ref, s0_ref, s1_ref, be_ref, loss_ref,
                 cnt_ref, gp_ref, i0s, i1s, r0s, r1s):
    b = pl.program_id(0)
    x = x_ref[...]
    lg = jnp.dot(x, wgn_ref[...], preferred_element_type=_F32) + bgn_ref[...]
    logits = lg[:, :E]
    sp = _softplus(lg[:, E:])
    noisy = logits + nz_ref[...] * sp

    ie = lax.broadcasted_iota(_I32, (_TB, E), 1)
    m1 = jnp.max(noisy, axis=1, keepdims=True)
    i0 = jnp.min(jnp.where(noisy == m1, ie, E), axis=1, keepdims=True)
    noisy2 = jnp.where(ie == i0, -jnp.inf, noisy)
    m2 = jnp.max(noisy2, axis=1, keepdims=True)
    i1 = jnp.min(jnp.where(noisy2 == m2, ie, E), axis=1, keepdims=True)

    d = jnp.exp(m2 - m1)
    w0 = 1.0 / (1.0 + d)
    w1 = 1.0 - w0

    a0 = (ie == i0).astype(_F32)
    a1 = (ie == i1).astype(_F32)
    s = a0 + a1

    gmax = jnp.max(logits, axis=1, keepdims=True)
    ge = jnp.exp(logits - gmax)
    gp = ge / jnp.sum(ge, axis=1, keepdims=True)

    @pl.when(b == 0)
    def _():
        cnt_ref[...] = jnp.zeros_like(cnt_ref)
        gp_ref[...] = jnp.zeros_like(gp_ref)

    carry = cnt_ref[...]
    it = lax.broadcasted_iota(_I32, (_TB, _TB), 0)
    iu = lax.broadcasted_iota(_I32, (_TB, _TB), 1)
    lstrict = (iu < it).astype(_F32)
    p = jnp.dot(lstrict, s, preferred_element_type=_F32,
                precision=lax.Precision.HIGHEST) + carry
    r0 = jnp.sum(p * a0, axis=1, keepdims=True)
    r1 = jnp.sum(p * a1, axis=1, keepdims=True)

    cnt = carry + jnp.sum(s, axis=0, keepdims=True)
    cnt_ref[...] = cnt
    gp_ref[...] += jnp.sum(gp, axis=0, keepdims=True)
    tb = pl.ds(b * _TB, _TB)
    i0s[tb, :] = i0
    i1s[tb, :] = i1
    r0s[tb, :] = r0.astype(_I32)
    r1s[tb, :] = r1.astype(_I32)
    w0_ref[...] = w0
    w1_ref[...] = w1

    @pl.when(b == _NTB - 1)
    def _():
        nbk = jnp.floor((cnt + (BS - 1)) * (1.0 / BS))
        ie1 = lax.broadcasted_iota(_I32, (E, E), 0)
        ie2 = lax.broadcasted_iota(_I32, (E, E), 1)
        ltri = (ie1 < ie2).astype(_F32)
        blkbase = jnp.dot(nbk, ltri, preferred_element_type=_F32,
                          precision=lax.Precision.HIGHEST)
        padbase = blkbase * float(BS)

        iet = lax.broadcasted_iota(_I32, (T, E), 1)
        a0t = (iet == i0s[...]).astype(_F32)
        a1t = (iet == i1s[...]).astype(_F32)
        pb0 = jnp.sum(a0t * padbase, axis=1, keepdims=True)
        pb1 = jnp.sum(a1t * padbase, axis=1, keepdims=True)
        s0_ref[...] = r0s[...] + pb0.astype(_I32)
        s1_ref[...] = r1s[...] + pb1.astype(_I32)

        used = jnp.sum(nbk)
        ib = lax.broadcasted_iota(_I32, (1, NB), 1).astype(_F32)
        ibc = jnp.minimum(ib, used - 1.0)
        owns = (jnp.broadcast_to(blkbase.reshape(E, 1), (E, NB))
                <= jnp.broadcast_to(ibc, (E, NB)))
        bex = jnp.sum(owns.astype(_F32), axis=0, keepdims=True) - 1.0
        flag = (ib < used)
        be_ref[...] = jnp.concatenate(
            [bex.astype(_I32), flag.astype(_I32), ibc.astype(_I32)], axis=0)
        loss_ref[...] = jnp.sum(cnt * gp_ref[...]).reshape(1, 1) * (
            float(E) / (float(NA) * float(T)))


def _gating(xf, nz, Wg, bg, Wn, bn, interpret=False):
    v1 = pl.BlockSpec((T // _NTB, 1), lambda i: (i, 0))
    full = pl.BlockSpec((T, 1), lambda i: (0, 0))
    out = pl.pallas_call(
        _gating_body,
        grid=(_NTB,),
        in_specs=[
            pl.BlockSpec((_TB, D), lambda i: (i, 0)),
            pl.BlockSpec((_TB, E), lambda i: (i, 0)),
            pl.BlockSpec((D, 2 * E), lambda i: (0, 0)),
            pl.BlockSpec((1, 2 * E), lambda i: (0, 0)),
        ],
        out_specs=[v1, v1, full, full,
                   pl.BlockSpec((3, NB), lambda i: (0, 0)),
                   pl.BlockSpec((1, 1), lambda i: (0, 0)),
                   pl.BlockSpec((1, E), lambda i: (0, 0)),
                   pl.BlockSpec((1, E), lambda i: (0, 0))],
        out_shape=[
            jax.ShapeDtypeStruct((T, 1), _F32),
            jax.ShapeDtypeStruct((T, 1), _F32),
            jax.ShapeDtypeStruct((T, 1), _I32),
            jax.ShapeDtypeStruct((T, 1), _I32),
            jax.ShapeDtypeStruct((3, NB), _I32),
            jax.ShapeDtypeStruct((1, 1), _F32),
            jax.ShapeDtypeStruct((1, E), _F32),
            jax.ShapeDtypeStruct((1, E), _F32),
        ],
        scratch_shapes=[
            pltpu.VMEM((T, 1), _I32),
            pltpu.VMEM((T, 1), _I32),
            pltpu.VMEM((T, 1), _I32),
            pltpu.VMEM((T, 1), _I32),
        ],
        interpret=interpret,
    )(xf, nz, jnp.concatenate([Wg, Wn], axis=1),
      jnp.concatenate([bg, bn]).reshape(1, 2 * E))
    return out[:6]


def _ffn_body(be_ref, xs_ref, ws_ref, w1_ref, b1_ref, w2_ref, b2_ref,
              ys_ref):
    @pl.when(be_ref[1, pl.program_id(0)] == 1)
    def _():
        xb = xs_ref[...]
        h = jnp.dot(xb, w1_ref[0], preferred_element_type=_F32) + b1_ref[0]
        h = 0.5 * h * (1.0 + lax.erf(h * 0.7071067811865476))
        y = jnp.dot(h, w2_ref[0], preferred_element_type=_F32) + b2_ref[0]
        ys_ref[...] = y * ws_ref[...]


def _ffn(be, xs, ws, W1, b1, W2, b2, interpret=False):
    grid_spec = pltpu.PrefetchScalarGridSpec(
        num_scalar_prefetch=1,
        grid=(NB,),
    in_specs=[
            pl.BlockSpec((BS, D), lambda b, be: (be[2, b], 0)),
            pl.BlockSpec((BS, 1), lambda b, be: (be[2, b], 0)),
            pl.BlockSpec((1, D, H), lambda b, be: (be[0, b], 0, 0)),
            pl.BlockSpec((1, 1, H), lambda b, be: (be[0, b], 0, 0)),
            pl.BlockSpec((1, H, D), lambda b, be: (be[0, b], 0, 0)),
            pl.BlockSpec((1, 1, D), lambda b, be: (be[0, b], 0, 0)),
        ],
        out_specs=pl.BlockSpec((BS, D), lambda b, be: (be[2, b], 0)),
    )
    return pl.pallas_call(
        _ffn_body,
        grid_spec=grid_spec,
        out_shape=jax.ShapeDtypeStruct((NPAD, D), _F32),
        interpret=interpret,
    )(be, xs, ws, W1, b1.reshape(E, 1, H), W2, b2.reshape(E, 1, D))


_NW = 32
_CHUNK = 64


_TPW = T // _NW


def _dispatch_body(xf_h, s0_h, s1_h, w0_h, w1_h, xs_h, ws_h,
                   i0_v, i1_v, wv0, wv1, rows_v,
                   sem_m, sem_r, sem_s):
    wid = lax.axis_index("s") * 2 + lax.axis_index("c")
    base = wid * _TPW
    sl = pl.ds(base, _TPW)
    meta = [pltpu.async_copy(s0_h.at[sl], i0_v.at[0], sem_m),
            pltpu.async_copy(s1_h.at[sl], i1_v.at[0], sem_m),
            pltpu.async_copy(w0_h.at[sl], wv0.at[0], sem_m),
            pltpu.async_copy(w1_h.at[sl], wv1.at[0], sem_m)]
    r0 = pltpu.async_copy(xf_h.at[sl], rows_v, sem_r)
    for m in meta:
        m.wait()
    r0.wait()
    scat = [pltpu.async_copy(rows_v, xs_h.at[i0_v.at[0]], sem_s),
            pltpu.async_copy(rows_v, xs_h.at[i1_v.at[0]], sem_s),
            pltpu.async_copy(wv0.at[0], ws_h.at[i0_v.at[0]], sem_s),
            pltpu.async_copy(wv1.at[0], ws_h.at[i1_v.at[0]], sem_s)]
    for s in scat:
        s.wait()


def _dispatch():
    return pl.kernel(
        _dispatch_body,
        out_type=[
            jax.ShapeDtypeStruct((NPAD, D), _F32),
            jax.ShapeDtypeStruct((NPAD,), _F32),
        ],
        mesh=plsc.VectorSubcoreMesh(core_axis_name="c",
                                    subcore_axis_name="s"),
        scratch_types=[
            pltpu.VMEM((1, _TPW), _I32),
            pltpu.VMEM((1, _TPW), _I32),
            pltpu.VMEM((1, _TPW), _F32),
            pltpu.VMEM((1, _TPW), _F32),
            pltpu.VMEM((_TPW, D), _F32),
            pltpu.SemaphoreType.DMA,
            pltpu.SemaphoreType.DMA,
            pltpu.SemaphoreType.DMA,
        ],
    )


def _combine_body(s0_h, s1_h, ys_h, out_h, i0_v, i1_v, acc_v, r1_v,
                  sem_m, sem_g, sem_o):
    wid = lax.axis_index("s") * 2 + lax.axis_index("c")
    base = wid * (T // _NW)
    nc = T // (_NW * _CHUNK)
    meta = []
    for c in range(nc):
        sl = pl.ds(base + c * _CHUNK, _CHUNK)
        meta += [pltpu.async_copy(s0_h.at[sl], i0_v.at[c], sem_m),
                 pltpu.async_copy(s1_h.at[sl], i1_v.at[c], sem_m)]
    for m in meta:
        m.wait()

    def _addrows(lo):
        def _addrow(j, carry):
            for dd in range(D // 16):
                sl = pl.ds(dd * 16, 16)
                acc_v[j, sl] = acc_v[j, sl] + r1_v[j, sl]
            return carry
        lax.fori_loop(lo, lo + _CHUNK // 2, _addrow, 0)

    for c in range(nc):
        g0 = pltpu.async_copy(ys_h.at[i0_v.at[c]], acc_v, sem_g)
        g1 = pltpu.async_copy(ys_h.at[i1_v.at[c]], r1_v, sem_g)
        g0.wait()
        g1.wait()
        _addrows(0)
        _addrows(_CHUNK // 2)
        pltpu.async_copy(acc_v, out_h.at[pl.ds(base + c * _CHUNK,
                                               _CHUNK)], sem_o).wait()


def _combine():
    return pl.kernel(
        _combine_body,
        out_type=jax.ShapeDtypeStruct((T, D), _F32),
        mesh=plsc.VectorSubcoreMesh(core_axis_name="c",
                                    subcore_axis_name="s"),
        scratch_types=[
            pltpu.VMEM((2, _CHUNK), _I32),
            pltpu.VMEM((2, _CHUNK), _I32),
            pltpu.VMEM((_CHUNK, D), _F32),
            pltpu.VMEM((_CHUNK, D), _F32),
            pltpu.SemaphoreType.DMA,
            pltpu.SemaphoreType.DMA,
            pltpu.SemaphoreType.DMA,
        ],
    )


def kernel(x, Wg, bg, Wn, bn, W1, b1, W2, b2):
    xf = x.reshape(T, D)
    nz = jax.random.normal(jax.random.key(42), (B, S, E),
                           dtype=_F32).reshape(T, E)
    w0, w1, s0, s1, be, loss = _gating(xf, nz, Wg, bg, Wn, bn)
    s0f = s0.reshape(T)
    s1f = s1.reshape(T)
    xs, ws = _dispatch()(xf, s0f, s1f, w0.reshape(T), w1.reshape(T))
    ys = _ffn(be, xs, ws.reshape(NPAD, 1), W1, b1, W2, b2)
    outf = _combine()(s0f, s1f, ys)
    return outf.reshape(B, S, D), loss.reshape(())
